# baseline (speedup 1.0000x reference)
import jax
import jax.numpy as jnp
from jax.experimental import pallas as pl
from jax.experimental.pallas import tpu as pltpu

HIDDEN_DIM = 2048
N_EXPERTS = 16
K = 2


def _router_kernel(x_ref, w_ref, logits_ref, probs_ref, idx_ref):
    x = x_ref[...]
    w = w_ref[...]
    logits = jnp.dot(x, w, preferred_element_type=jnp.float32)
    logits_ref[...] = logits

    m = jnp.max(logits, axis=-1, keepdims=True)
    e = jnp.exp(logits - m)
    probs = e / jnp.sum(e, axis=-1, keepdims=True)

    cols = jax.lax.broadcasted_iota(jnp.int32, probs.shape, 1)
    i1 = jnp.argmax(probs, axis=-1)
    p1 = jnp.max(probs, axis=-1)
    masked = jnp.where(cols == i1[:, None], -jnp.inf, probs)
    i2 = jnp.argmax(masked, axis=-1)
    p2 = jnp.max(masked, axis=-1)

    s = p1 + p2 + 1e-8
    kcols = jax.lax.broadcasted_iota(jnp.int32, (x.shape[0], K), 1)
    probs_ref[...] = jnp.where(kcols == 0, (p1 / s)[:, None], (p2 / s)[:, None])
    idx_ref[...] = jnp.where(kcols == 0, i1[:, None], i2[:, None])


def kernel(hidden_states, gate_weight):
    B, S, H = hidden_states.shape
    T = B * S
    x = hidden_states.reshape(T, H)
    wt = gate_weight.astype(hidden_states.dtype).T

    TILE = 1024
    grid = (T // TILE,)

    logits, probs, idx = pl.pallas_call(
        _router_kernel,
        grid=grid,
        in_specs=[
            pl.BlockSpec((TILE, H), lambda i: (i, 0)),
            pl.BlockSpec((H, N_EXPERTS), lambda i: (0, 0)),
        ],
        out_specs=[
            pl.BlockSpec((TILE, N_EXPERTS), lambda i: (i, 0)),
            pl.BlockSpec((TILE, K), lambda i: (i, 0)),
            pl.BlockSpec((TILE, K), lambda i: (i, 0)),
        ],
        out_shape=[
            jax.ShapeDtypeStruct((T, N_EXPERTS), jnp.float32),
            jax.ShapeDtypeStruct((T, K), jnp.float32),
            jax.ShapeDtypeStruct((T, K), jnp.int32),
        ],
        compiler_params=pltpu.CompilerParams(
            dimension_semantics=("arbitrary",),
        ),
    )(x, wt)

    return (
        probs.reshape(B, S, K),
        idx.reshape(B, S, K),
        logits.reshape(B, S, N_EXPERTS),
    )


# speedup vs baseline: 1.0111x; 1.0111x over previous
import jax
import jax.numpy as jnp
from jax.experimental import pallas as pl
from jax.experimental.pallas import tpu as pltpu

HIDDEN_DIM = 2048
N_EXPERTS = 16
K = 2


def _router_kernel(x_ref, w_ref, logits_ref, probs_ref, idx_ref):
    x = x_ref[...]
    w = w_ref[...]
    logits = jnp.dot(x, w, preferred_element_type=jnp.float32)
    logits_ref[...] = logits

    m = jnp.max(logits, axis=-1, keepdims=True)
    e = jnp.exp(logits - m)
    probs = e / jnp.sum(e, axis=-1, keepdims=True)

    cols = jax.lax.broadcasted_iota(jnp.int32, probs.shape, 1)
    i1 = jnp.argmax(probs, axis=-1)
    p1 = jnp.max(probs, axis=-1)
    masked = jnp.where(cols == i1[:, None], -jnp.inf, probs)
    i2 = jnp.argmax(masked, axis=-1)
    p2 = jnp.max(masked, axis=-1)

    s = p1 + p2 + 1e-8
    kcols = jax.lax.broadcasted_iota(jnp.int32, (x.shape[0], K), 1)
    probs_ref[...] = jnp.where(kcols == 0, (p1 / s)[:, None], (p2 / s)[:, None])
    idx_ref[...] = jnp.where(kcols == 0, i1[:, None], i2[:, None])


def kernel(hidden_states, gate_weight):
    B, S, H = hidden_states.shape
    T = B * S
    x = hidden_states.reshape(T, H)
    wt = gate_weight.astype(hidden_states.dtype).T

    TILE = 1024
    grid = (T // TILE,)

    logits, probs, idx = pl.pallas_call(
        _router_kernel,
        grid=grid,
        in_specs=[
            pl.BlockSpec((TILE, H), lambda i: (i, 0)),
            pl.BlockSpec((H, N_EXPERTS), lambda i: (0, 0)),
        ],
        out_specs=[
            pl.BlockSpec((TILE, N_EXPERTS), lambda i: (i, 0)),
            pl.BlockSpec((TILE, K), lambda i: (i, 0)),
            pl.BlockSpec((TILE, K), lambda i: (i, 0)),
        ],
        out_shape=[
            jax.ShapeDtypeStruct((T, N_EXPERTS), jnp.float32),
            jax.ShapeDtypeStruct((T, K), jnp.float32),
            jax.ShapeDtypeStruct((T, K), jnp.int32),
        ],
        compiler_params=pltpu.CompilerParams(
            dimension_semantics=("parallel",),
        ),
    )(x, wt)

    return (
        probs.reshape(B, S, K),
        idx.reshape(B, S, K),
        logits.reshape(B, S, N_EXPERTS),
    )
